# Initial kernel scaffold; baseline (speedup 1.0000x reference)
import functools

import jax
import jax.numpy as jnp
from jax import lax
from jax.experimental import pallas as pl
from jax.experimental.pallas import tpu as pltpu
import jax.experimental.pallas.tpu_sc as plsc

N = 10000
E = 320000
F_IN = 128
DIM = 10
DPAD = 16

NW = 32
EPW = E // NW
C = 125
NCHUNK = EPW // C
RPT = N // 16

BLK = 1000
GRID = N // BLK


def _pad_mat(w, rows, cols):
    return jnp.zeros((rows, cols), jnp.float32).at[: w.shape[0], : w.shape[1]].set(w)


def _pad_vec(b, cols):
    return jnp.zeros((1, cols), jnp.float32).at[0, : b.shape[0]].set(b)


_MESH = plsc.VectorSubcoreMesh(core_axis_name="c", subcore_axis_name="s")


@functools.partial(
    pl.kernel,
    out_type=jax.ShapeDtypeStruct((2, N, DPAD), jnp.float32),
    mesh=_MESH,
    scratch_types=[
        pltpu.VMEM((NCHUNK, C), jnp.int32),
        pltpu.VMEM((NCHUNK, C), jnp.int32),
        pltpu.VMEM((C, DPAD), jnp.float32),
        pltpu.VMEM((C, DPAD), jnp.float32),
        pltpu.VMEM_SHARED((N, DPAD), jnp.float32),
        pltpu.SemaphoreType.DMA,
    ],
)
def _sc_agg(table_hbm, src_hbm, dst_hbm, parts_hbm,
            src_v, dst_v, rows_v, zero_v, acc_sh, sem):
    c = lax.axis_index("c")
    s = lax.axis_index("s")
    wid = c * 16 + s

    pltpu.sync_copy(src_hbm.at[wid], src_v)
    pltpu.sync_copy(dst_hbm.at[wid], dst_v)

    def _zero_row(i, _):
        zero_v[i, :] = jnp.zeros((DPAD,), jnp.float32)
        return 0
    lax.fori_loop(0, C, _zero_row, 0)

    def _zero_stripe(j, _):
        pltpu.sync_copy(zero_v, acc_sh.at[pl.ds(s * RPT + j * C, C)])
        return 0
    lax.fori_loop(0, RPT // C, _zero_stripe, 0)

    plsc.subcore_barrier()

    def _edges(j, _):
        pltpu.async_copy(table_hbm.at[src_v.at[j]], rows_v, sem).wait()
        pltpu.sync_copy(rows_v, acc_sh.at[dst_v.at[j]], add=True)
        return 0
    lax.fori_loop(0, NCHUNK, _edges, 0)

    plsc.subcore_barrier()
    pltpu.sync_copy(acc_sh.at[pl.ds(s * RPT, RPT)],
                    parts_hbm.at[c, pl.ds(s * RPT, RPT)])


def _proj_body(x_ref, w_ref, o_ref):
    o_ref[...] = jnp.dot(x_ref[...], w_ref[...],
                         preferred_element_type=jnp.float32)


_proj = pl.pallas_call(
    _proj_body,
    grid=(GRID,),
    in_specs=[
        pl.BlockSpec((BLK, F_IN), lambda k: (k, 0)),
        pl.BlockSpec((F_IN, DPAD), lambda k: (0, 0)),
    ],
    out_specs=pl.BlockSpec((BLK, DPAD), lambda k: (k, 0)),
    out_shape=jax.ShapeDtypeStruct((N, DPAD), jnp.float32),
)


def _mlp_body(u_ref, parts_ref, ba_ref, wb_ref, bb_ref, wa2_ref,
              unext_ref, s_ref):
    k = pl.program_id(0)
    pre = u_ref[...] + parts_ref[0] + parts_ref[1] + ba_ref[...]
    h = jnp.maximum(pre, 0.0)
    t = jnp.dot(h, wb_ref[...], preferred_element_type=jnp.float32) + bb_ref[...]
    xv = jnp.maximum(t, 0.0)
    unext_ref[...] = jnp.dot(xv, wa2_ref[...],
                             preferred_element_type=jnp.float32)

    @pl.when(k == 0)
    def _():
        s_ref[...] = jnp.zeros_like(s_ref)

    s_ref[...] += jnp.sum(xv, axis=0, keepdims=True)


_mlp = pl.pallas_call(
    _mlp_body,
    grid=(GRID,),
    in_specs=[
        pl.BlockSpec((BLK, DPAD), lambda k: (k, 0)),
        pl.BlockSpec((2, BLK, DPAD), lambda k: (0, k, 0)),
        pl.BlockSpec((1, DPAD), lambda k: (0, 0)),
        pl.BlockSpec((DPAD, DPAD), lambda k: (0, 0)),
        pl.BlockSpec((1, DPAD), lambda k: (0, 0)),
        pl.BlockSpec((DPAD, DPAD), lambda k: (0, 0)),
    ],
    out_specs=[
        pl.BlockSpec((BLK, DPAD), lambda k: (k, 0)),
        pl.BlockSpec((1, DPAD), lambda k: (0, 0)),
    ],
    out_shape=[
        jax.ShapeDtypeStruct((N, DPAD), jnp.float32),
        jax.ShapeDtypeStruct((1, DPAD), jnp.float32),
    ],
)


def _mlp_last_body(u_ref, parts_ref, ba_ref, wb_ref, bb_ref, s_ref):
    k = pl.program_id(0)
    pre = u_ref[...] + parts_ref[0] + parts_ref[1] + ba_ref[...]
    h = jnp.maximum(pre, 0.0)
    t = jnp.dot(h, wb_ref[...], preferred_element_type=jnp.float32) + bb_ref[...]
    xv = jnp.maximum(t, 0.0)

    @pl.when(k == 0)
    def _():
        s_ref[...] = jnp.zeros_like(s_ref)

    s_ref[...] += jnp.sum(xv, axis=0, keepdims=True)


_mlp_last = pl.pallas_call(
    _mlp_last_body,
    grid=(GRID,),
    in_specs=[
        pl.BlockSpec((BLK, DPAD), lambda k: (k, 0)),
        pl.BlockSpec((2, BLK, DPAD), lambda k: (0, k, 0)),
        pl.BlockSpec((1, DPAD), lambda k: (0, 0)),
        pl.BlockSpec((DPAD, DPAD), lambda k: (0, 0)),
        pl.BlockSpec((1, DPAD), lambda k: (0, 0)),
    ],
    out_specs=pl.BlockSpec((1, DPAD), lambda k: (0, 0)),
    out_shape=jax.ShapeDtypeStruct((1, DPAD), jnp.float32),
)


def _final_body(s_ref, l_ref, o_ref):
    tot = jnp.sum(s_ref[...] * l_ref[...]) * (1.0 / N)
    o_ref[...] = jax.nn.sigmoid(tot).reshape(1, 1)


_final = pl.pallas_call(
    _final_body,
    in_specs=[
        pl.BlockSpec((8, DPAD), lambda: (0, 0)),
        pl.BlockSpec((8, DPAD), lambda: (0, 0)),
    ],
    out_specs=pl.BlockSpec((1, 1), lambda: (0, 0)),
    out_shape=jax.ShapeDtypeStruct((1, 1), jnp.float32),
)


def kernel(x, edge_index,
           W1a, b1a, W1b, b1b,
           W2a, b2a, W2b, b2b,
           W3a, b3a, W3b, b3b,
           W4a, b4a, W4b, b4b,
           W5a, b5a, W5b, b5b,
           L1, L2, L3, L4, L5):
    src3 = edge_index[0].reshape(NW, NCHUNK, C)
    dst3 = edge_index[1].reshape(NW, NCHUNK, C)

    W1a_p = _pad_mat(W1a, F_IN, DPAD)
    Was = [None, _pad_mat(W2a, DPAD, DPAD), _pad_mat(W3a, DPAD, DPAD),
           _pad_mat(W4a, DPAD, DPAD), _pad_mat(W5a, DPAD, DPAD)]
    Wbs = [_pad_mat(W1b, DPAD, DPAD), _pad_mat(W2b, DPAD, DPAD),
           _pad_mat(W3b, DPAD, DPAD), _pad_mat(W4b, DPAD, DPAD),
           _pad_mat(W5b, DPAD, DPAD)]
    bas = [_pad_vec(b1a, DPAD), _pad_vec(b2a, DPAD), _pad_vec(b3a, DPAD),
           _pad_vec(b4a, DPAD), _pad_vec(b5a, DPAD)]
    bbs = [_pad_vec(b1b, DPAD), _pad_vec(b2b, DPAD), _pad_vec(b3b, DPAD),
           _pad_vec(b4b, DPAD), _pad_vec(b5b, DPAD)]

    u = _proj(x, W1a_p)
    sums = []
    for i in range(5):
        parts = _sc_agg(u, src3, dst3)
        if i < 4:
            u, s_i = _mlp(u, parts, bas[i], Wbs[i], bbs[i], Was[i + 1])
        else:
            s_i = _mlp_last(u, parts, bas[i], Wbs[i], bbs[i])
        sums.append(s_i)

    S = jnp.concatenate(sums + [jnp.zeros((3, DPAD), jnp.float32)], axis=0)
    Lrows = jnp.stack([_pad_vec(L[:, 0], DPAD)[0]
                       for L in (L1, L2, L3, L4, L5)], axis=0)
    Lp = jnp.concatenate([Lrows, jnp.zeros((3, DPAD), jnp.float32)], axis=0)
    return _final(S, Lp)


# speedup vs baseline: 12.4347x; 12.4347x over previous
import functools

import jax
import jax.numpy as jnp
from jax import lax
from jax.experimental import pallas as pl
from jax.experimental.pallas import tpu as pltpu
import jax.experimental.pallas.tpu_sc as plsc

N = 10000
E = 320000
F_IN = 128
DIM = 10
DPAD = 16

NW = 32
EPW = E // NW
C = 125
NCHUNK = EPW // C
NACC = 10240
RPT = NACC // 16
ZC = 128

BLK = 1000
GRID = N // BLK


def _pad_mat(w, rows, cols):
    return jnp.zeros((rows, cols), jnp.float32).at[: w.shape[0], : w.shape[1]].set(w)


def _pad_vec(b, cols):
    return jnp.zeros((1, cols), jnp.float32).at[0, : b.shape[0]].set(b)


_MESH = plsc.VectorSubcoreMesh(core_axis_name="c", subcore_axis_name="s")


@functools.partial(
    pl.kernel,
    out_type=jax.ShapeDtypeStruct((2, NACC, DPAD), jnp.float32),
    mesh=_MESH,
    scratch_types=[
        pltpu.VMEM((NCHUNK, C), jnp.int32),
        pltpu.VMEM((NCHUNK, C), jnp.int32),
        pltpu.VMEM((C, DPAD), jnp.float32),
        pltpu.VMEM((ZC, DPAD), jnp.float32),
        pltpu.VMEM_SHARED((NACC, DPAD), jnp.float32),
        pltpu.SemaphoreType.DMA,
    ],
    compiler_params=pltpu.CompilerParams(use_tc_tiling_on_sc=False),
)
def _sc_agg(table_hbm, src_hbm, dst_hbm, parts_hbm,
            src_v, dst_v, rows_v, zero_v, acc_sh, sem):
    c = lax.axis_index("c")
    s = lax.axis_index("s")
    wid = c * 16 + s

    pltpu.sync_copy(src_hbm.at[wid], src_v)
    pltpu.sync_copy(dst_hbm.at[wid], dst_v)

    def _zero_row(i, _):
        zero_v[i, :] = jnp.zeros((DPAD,), jnp.float32)
        return 0
    lax.fori_loop(0, ZC, _zero_row, 0)

    def _zero_stripe(j, _):
        pltpu.sync_copy(zero_v, acc_sh.at[pl.ds(s * RPT + j * ZC, ZC)])
        return 0
    lax.fori_loop(0, RPT // ZC, _zero_stripe, 0)

    plsc.subcore_barrier()

    def _edges(j, _):
        pltpu.async_copy(table_hbm.at[src_v.at[j]], rows_v, sem).wait()
        pltpu.sync_copy(rows_v, acc_sh.at[dst_v.at[j]], add=True)
        return 0
    lax.fori_loop(0, NCHUNK, _edges, 0)

    plsc.subcore_barrier()
    pltpu.sync_copy(acc_sh.at[pl.ds(s * RPT, RPT)],
                    parts_hbm.at[c, pl.ds(s * RPT, RPT)])


def _proj_body(x_ref, w_ref, o_ref):
    o_ref[...] = jnp.dot(x_ref[...], w_ref[...],
                         preferred_element_type=jnp.float32)


_proj = pl.pallas_call(
    _proj_body,
    grid=(GRID,),
    in_specs=[
        pl.BlockSpec((BLK, F_IN), lambda k: (k, 0)),
        pl.BlockSpec((F_IN, DPAD), lambda k: (0, 0)),
    ],
    out_specs=pl.BlockSpec((BLK, DPAD), lambda k: (k, 0)),
    out_shape=jax.ShapeDtypeStruct((N, DPAD), jnp.float32),
)


def _mlp_body(u_ref, parts_ref, ba_ref, wb_ref, bb_ref, wa2_ref,
              unext_ref, s_ref):
    k = pl.program_id(0)
    pre = u_ref[...] + parts_ref[0] + parts_ref[1] + ba_ref[...]
    h = jnp.maximum(pre, 0.0)
    t = jnp.dot(h, wb_ref[...], preferred_element_type=jnp.float32) + bb_ref[...]
    xv = jnp.maximum(t, 0.0)
    unext_ref[...] = jnp.dot(xv, wa2_ref[...],
                             preferred_element_type=jnp.float32)

    @pl.when(k == 0)
    def _():
        s_ref[...] = jnp.zeros_like(s_ref)

    s_ref[...] += jnp.sum(xv, axis=0, keepdims=True)


_mlp = pl.pallas_call(
    _mlp_body,
    grid=(GRID,),
    in_specs=[
        pl.BlockSpec((BLK, DPAD), lambda k: (k, 0)),
        pl.BlockSpec((2, BLK, DPAD), lambda k: (0, k, 0)),
        pl.BlockSpec((1, DPAD), lambda k: (0, 0)),
        pl.BlockSpec((DPAD, DPAD), lambda k: (0, 0)),
        pl.BlockSpec((1, DPAD), lambda k: (0, 0)),
        pl.BlockSpec((DPAD, DPAD), lambda k: (0, 0)),
    ],
    out_specs=[
        pl.BlockSpec((BLK, DPAD), lambda k: (k, 0)),
        pl.BlockSpec((1, DPAD), lambda k: (0, 0)),
    ],
    out_shape=[
        jax.ShapeDtypeStruct((N, DPAD), jnp.float32),
        jax.ShapeDtypeStruct((1, DPAD), jnp.float32),
    ],
)


def _mlp_last_body(u_ref, parts_ref, ba_ref, wb_ref, bb_ref, s_ref):
    k = pl.program_id(0)
    pre = u_ref[...] + parts_ref[0] + parts_ref[1] + ba_ref[...]
    h = jnp.maximum(pre, 0.0)
    t = jnp.dot(h, wb_ref[...], preferred_element_type=jnp.float32) + bb_ref[...]
    xv = jnp.maximum(t, 0.0)

    @pl.when(k == 0)
    def _():
        s_ref[...] = jnp.zeros_like(s_ref)

    s_ref[...] += jnp.sum(xv, axis=0, keepdims=True)


_mlp_last = pl.pallas_call(
    _mlp_last_body,
    grid=(GRID,),
    in_specs=[
        pl.BlockSpec((BLK, DPAD), lambda k: (k, 0)),
        pl.BlockSpec((2, BLK, DPAD), lambda k: (0, k, 0)),
        pl.BlockSpec((1, DPAD), lambda k: (0, 0)),
        pl.BlockSpec((DPAD, DPAD), lambda k: (0, 0)),
        pl.BlockSpec((1, DPAD), lambda k: (0, 0)),
    ],
    out_specs=pl.BlockSpec((1, DPAD), lambda k: (0, 0)),
    out_shape=jax.ShapeDtypeStruct((1, DPAD), jnp.float32),
)


def _final_body(s_ref, l_ref, o_ref):
    tot = jnp.sum(s_ref[...] * l_ref[...]) * (1.0 / N)
    o_ref[...] = jax.nn.sigmoid(tot).reshape(1, 1)


_final = pl.pallas_call(
    _final_body,
    in_specs=[
        pl.BlockSpec((8, DPAD), lambda: (0, 0)),
        pl.BlockSpec((8, DPAD), lambda: (0, 0)),
    ],
    out_specs=pl.BlockSpec((1, 1), lambda: (0, 0)),
    out_shape=jax.ShapeDtypeStruct((1, 1), jnp.float32),
)


def kernel(x, edge_index,
           W1a, b1a, W1b, b1b,
           W2a, b2a, W2b, b2b,
           W3a, b3a, W3b, b3b,
           W4a, b4a, W4b, b4b,
           W5a, b5a, W5b, b5b,
           L1, L2, L3, L4, L5):
    src3 = edge_index[0].reshape(NW, NCHUNK, C)
    dst3 = edge_index[1].reshape(NW, NCHUNK, C)

    W1a_p = _pad_mat(W1a, F_IN, DPAD)
    Was = [None, _pad_mat(W2a, DPAD, DPAD), _pad_mat(W3a, DPAD, DPAD),
           _pad_mat(W4a, DPAD, DPAD), _pad_mat(W5a, DPAD, DPAD)]
    Wbs = [_pad_mat(W1b, DPAD, DPAD), _pad_mat(W2b, DPAD, DPAD),
           _pad_mat(W3b, DPAD, DPAD), _pad_mat(W4b, DPAD, DPAD),
           _pad_mat(W5b, DPAD, DPAD)]
    bas = [_pad_vec(b1a, DPAD), _pad_vec(b2a, DPAD), _pad_vec(b3a, DPAD),
           _pad_vec(b4a, DPAD), _pad_vec(b5a, DPAD)]
    bbs = [_pad_vec(b1b, DPAD), _pad_vec(b2b, DPAD), _pad_vec(b3b, DPAD),
           _pad_vec(b4b, DPAD), _pad_vec(b5b, DPAD)]

    u = _proj(x, W1a_p)
    sums = []
    for i in range(5):
        parts = _sc_agg(u, src3, dst3)
        if i < 4:
            u, s_i = _mlp(u, parts, bas[i], Wbs[i], bbs[i], Was[i + 1])
        else:
            s_i = _mlp_last(u, parts, bas[i], Wbs[i], bbs[i])
        sums.append(s_i)

    S = jnp.concatenate(sums + [jnp.zeros((3, DPAD), jnp.float32)], axis=0)
    Lrows = jnp.stack([_pad_vec(L[:, 0], DPAD)[0]
                       for L in (L1, L2, L3, L4, L5)], axis=0)
    Lp = jnp.concatenate([Lrows, jnp.zeros((3, DPAD), jnp.float32)], axis=0)
    return _final(S, Lp)


# speedup vs baseline: 20.7801x; 1.6711x over previous
import functools

import jax
import jax.numpy as jnp
from jax import lax
from jax.experimental import pallas as pl
from jax.experimental.pallas import tpu as pltpu
import jax.experimental.pallas.tpu_sc as plsc

N = 10000
E = 320000
F_IN = 128
DIM = 10
DPAD = 16

NW = 32
EPW = E // NW
C = 125
NCHUNK = EPW // C
K = 8
G = NCHUNK // K
NACC = 10240
RPT = NACC // 16
ZC = 128

BLK = 1000
GRID = N // BLK


def _pad_mat(w, rows, cols):
    return jnp.zeros((rows, cols), jnp.float32).at[: w.shape[0], : w.shape[1]].set(w)


def _pad_vec(b, cols):
    return jnp.zeros((1, cols), jnp.float32).at[0, : b.shape[0]].set(b)


_MESH = plsc.VectorSubcoreMesh(core_axis_name="c", subcore_axis_name="s")


@functools.partial(
    pl.kernel,
    out_type=jax.ShapeDtypeStruct((2, NACC, DPAD), jnp.float32),
    mesh=_MESH,
    scratch_types=[
        pltpu.VMEM((NCHUNK, C), jnp.int32),
        pltpu.VMEM((NCHUNK, C), jnp.int32),
        pltpu.VMEM((K, C, DPAD), jnp.float32),
        pltpu.VMEM((ZC, DPAD), jnp.float32),
        pltpu.VMEM_SHARED((NACC, DPAD), jnp.float32),
        pltpu.SemaphoreType.DMA,
        pltpu.SemaphoreType.DMA,
    ],
    compiler_params=pltpu.CompilerParams(use_tc_tiling_on_sc=False),
)
def _sc_agg(table_hbm, src_hbm, dst_hbm, parts_hbm,
            src_v, dst_v, rows_v, zero_v, acc_sh, gsem, ssem):
    c = lax.axis_index("c")
    s = lax.axis_index("s")
    wid = c * 16 + s

    pltpu.sync_copy(src_hbm.at[wid], src_v)
    pltpu.sync_copy(dst_hbm.at[wid], dst_v)

    def _zero_row(i, _):
        zero_v[i, :] = jnp.zeros((DPAD,), jnp.float32)
        return 0
    lax.fori_loop(0, ZC, _zero_row, 0)

    def _zero_stripe(j, _):
        pltpu.sync_copy(zero_v, acc_sh.at[pl.ds(s * RPT + j * ZC, ZC)])
        return 0
    lax.fori_loop(0, RPT // ZC, _zero_stripe, 0)

    plsc.subcore_barrier()

    def _edges(g, _):
        base = g * K
        gds = [pltpu.async_copy(table_hbm.at[src_v.at[base + b]],
                                rows_v.at[b], gsem) for b in range(K)]
        for d in gds:
            d.wait()
        sds = [pltpu.async_copy(rows_v.at[b], acc_sh.at[dst_v.at[base + b]],
                                ssem, add=True) for b in range(K)]
        for d in sds:
            d.wait()
        return 0
    lax.fori_loop(0, G, _edges, 0)

    plsc.subcore_barrier()
    pltpu.sync_copy(acc_sh.at[pl.ds(s * RPT, RPT)],
                    parts_hbm.at[c, pl.ds(s * RPT, RPT)])


def _proj_body(x_ref, w_ref, o_ref):
    o_ref[...] = jnp.dot(x_ref[...], w_ref[...],
                         preferred_element_type=jnp.float32)


_proj = pl.pallas_call(
    _proj_body,
    grid=(GRID,),
    in_specs=[
        pl.BlockSpec((BLK, F_IN), lambda k: (k, 0)),
        pl.BlockSpec((F_IN, DPAD), lambda k: (0, 0)),
    ],
    out_specs=pl.BlockSpec((BLK, DPAD), lambda k: (k, 0)),
    out_shape=jax.ShapeDtypeStruct((N, DPAD), jnp.float32),
)


def _mlp_body(u_ref, parts_ref, ba_ref, wb_ref, bb_ref, wa2_ref,
              unext_ref, s_ref):
    k = pl.program_id(0)
    pre = u_ref[...] + parts_ref[0] + parts_ref[1] + ba_ref[...]
    h = jnp.maximum(pre, 0.0)
    t = jnp.dot(h, wb_ref[...], preferred_element_type=jnp.float32) + bb_ref[...]
    xv = jnp.maximum(t, 0.0)
    unext_ref[...] = jnp.dot(xv, wa2_ref[...],
                             preferred_element_type=jnp.float32)

    @pl.when(k == 0)
    def _():
        s_ref[...] = jnp.zeros_like(s_ref)

    s_ref[...] += jnp.sum(xv, axis=0, keepdims=True)


_mlp = pl.pallas_call(
    _mlp_body,
    grid=(GRID,),
    in_specs=[
        pl.BlockSpec((BLK, DPAD), lambda k: (k, 0)),
        pl.BlockSpec((2, BLK, DPAD), lambda k: (0, k, 0)),
        pl.BlockSpec((1, DPAD), lambda k: (0, 0)),
        pl.BlockSpec((DPAD, DPAD), lambda k: (0, 0)),
        pl.BlockSpec((1, DPAD), lambda k: (0, 0)),
        pl.BlockSpec((DPAD, DPAD), lambda k: (0, 0)),
    ],
    out_specs=[
        pl.BlockSpec((BLK, DPAD), lambda k: (k, 0)),
        pl.BlockSpec((1, DPAD), lambda k: (0, 0)),
    ],
    out_shape=[
        jax.ShapeDtypeStruct((N, DPAD), jnp.float32),
        jax.ShapeDtypeStruct((1, DPAD), jnp.float32),
    ],
)


def _mlp_last_body(u_ref, parts_ref, ba_ref, wb_ref, bb_ref, s_ref):
    k = pl.program_id(0)
    pre = u_ref[...] + parts_ref[0] + parts_ref[1] + ba_ref[...]
    h = jnp.maximum(pre, 0.0)
    t = jnp.dot(h, wb_ref[...], preferred_element_type=jnp.float32) + bb_ref[...]
    xv = jnp.maximum(t, 0.0)

    @pl.when(k == 0)
    def _():
        s_ref[...] = jnp.zeros_like(s_ref)

    s_ref[...] += jnp.sum(xv, axis=0, keepdims=True)


_mlp_last = pl.pallas_call(
    _mlp_last_body,
    grid=(GRID,),
    in_specs=[
        pl.BlockSpec((BLK, DPAD), lambda k: (k, 0)),
        pl.BlockSpec((2, BLK, DPAD), lambda k: (0, k, 0)),
        pl.BlockSpec((1, DPAD), lambda k: (0, 0)),
        pl.BlockSpec((DPAD, DPAD), lambda k: (0, 0)),
        pl.BlockSpec((1, DPAD), lambda k: (0, 0)),
    ],
    out_specs=pl.BlockSpec((1, DPAD), lambda k: (0, 0)),
    out_shape=jax.ShapeDtypeStruct((1, DPAD), jnp.float32),
)


def _final_body(s_ref, l_ref, o_ref):
    tot = jnp.sum(s_ref[...] * l_ref[...]) * (1.0 / N)
    o_ref[...] = jax.nn.sigmoid(tot).reshape(1, 1)


_final = pl.pallas_call(
    _final_body,
    in_specs=[
        pl.BlockSpec((8, DPAD), lambda: (0, 0)),
        pl.BlockSpec((8, DPAD), lambda: (0, 0)),
    ],
    out_specs=pl.BlockSpec((1, 1), lambda: (0, 0)),
    out_shape=jax.ShapeDtypeStruct((1, 1), jnp.float32),
)


def kernel(x, edge_index,
           W1a, b1a, W1b, b1b,
           W2a, b2a, W2b, b2b,
           W3a, b3a, W3b, b3b,
           W4a, b4a, W4b, b4b,
           W5a, b5a, W5b, b5b,
           L1, L2, L3, L4, L5):
    src3 = edge_index[0].reshape(NW, NCHUNK, C)
    dst3 = edge_index[1].reshape(NW, NCHUNK, C)

    W1a_p = _pad_mat(W1a, F_IN, DPAD)
    Was = [None, _pad_mat(W2a, DPAD, DPAD), _pad_mat(W3a, DPAD, DPAD),
           _pad_mat(W4a, DPAD, DPAD), _pad_mat(W5a, DPAD, DPAD)]
    Wbs = [_pad_mat(W1b, DPAD, DPAD), _pad_mat(W2b, DPAD, DPAD),
           _pad_mat(W3b, DPAD, DPAD), _pad_mat(W4b, DPAD, DPAD),
           _pad_mat(W5b, DPAD, DPAD)]
    bas = [_pad_vec(b1a, DPAD), _pad_vec(b2a, DPAD), _pad_vec(b3a, DPAD),
           _pad_vec(b4a, DPAD), _pad_vec(b5a, DPAD)]
    bbs = [_pad_vec(b1b, DPAD), _pad_vec(b2b, DPAD), _pad_vec(b3b, DPAD),
           _pad_vec(b4b, DPAD), _pad_vec(b5b, DPAD)]

    u = _proj(x, W1a_p)
    sums = []
    for i in range(5):
        parts = _sc_agg(u, src3, dst3)
        if i < 4:
            u, s_i = _mlp(u, parts, bas[i], Wbs[i], bbs[i], Was[i + 1])
        else:
            s_i = _mlp_last(u, parts, bas[i], Wbs[i], bbs[i])
        sums.append(s_i)

    S = jnp.concatenate(sums + [jnp.zeros((3, DPAD), jnp.float32)], axis=0)
    Lrows = jnp.stack([_pad_vec(L[:, 0], DPAD)[0]
                       for L in (L1, L2, L3, L4, L5)], axis=0)
    Lp = jnp.concatenate([Lrows, jnp.zeros((3, DPAD), jnp.float32)], axis=0)
    return _final(S, Lp)


# speedup vs baseline: 22.4520x; 1.0805x over previous
import functools

import jax
import jax.numpy as jnp
from jax import lax
from jax.experimental import pallas as pl
from jax.experimental.pallas import tpu as pltpu
import jax.experimental.pallas.tpu_sc as plsc

N = 10000
E = 320000
F_IN = 128
DIM = 10
DPAD = 16

NW = 32
EPW = E // NW
C = 125
NCHUNK = EPW // C
K = 8
G = NCHUNK // K
NACC = 10240
RPT = NACC // 16
ZC = 128

BLK = 1000
GRID = N // BLK


def _pad_mat(w, rows, cols):
    return jnp.zeros((rows, cols), jnp.float32).at[: w.shape[0], : w.shape[1]].set(w)


def _pad_vec(b, cols):
    return jnp.zeros((1, cols), jnp.float32).at[0, : b.shape[0]].set(b)


_MESH = plsc.VectorSubcoreMesh(core_axis_name="c", subcore_axis_name="s")


@functools.partial(
    pl.kernel,
    out_type=jax.ShapeDtypeStruct((2, NACC, DPAD), jnp.float32),
    mesh=_MESH,
    scratch_types=[
        pltpu.VMEM((NCHUNK, C), jnp.int32),
        pltpu.VMEM((NCHUNK, C), jnp.int32),
        pltpu.VMEM((2, K, C, DPAD), jnp.float32),
        pltpu.VMEM((ZC, DPAD), jnp.float32),
        pltpu.VMEM_SHARED((NACC, DPAD), jnp.float32),
        pltpu.SemaphoreType.DMA,
        pltpu.SemaphoreType.DMA,
        pltpu.SemaphoreType.DMA,
    ],
    compiler_params=pltpu.CompilerParams(use_tc_tiling_on_sc=False),
)
def _sc_agg(table_hbm, src_hbm, dst_hbm, parts_hbm,
            src_v, dst_v, rows_v, zero_v, acc_sh, gsemA, gsemB, ssem):
    c = lax.axis_index("c")
    s = lax.axis_index("s")
    wid = c * 16 + s

    pltpu.sync_copy(src_hbm.at[wid], src_v)
    pltpu.sync_copy(dst_hbm.at[wid], dst_v)

    def _zero_row(i, _):
        zero_v[i, :] = jnp.zeros((DPAD,), jnp.float32)
        return 0
    lax.fori_loop(0, ZC, _zero_row, 0)

    def _zero_stripe(j, _):
        pltpu.sync_copy(zero_v, acc_sh.at[pl.ds(s * RPT + j * ZC, ZC)])
        return 0
    lax.fori_loop(0, RPT // ZC, _zero_stripe, 0)

    plsc.subcore_barrier()

    def _pair(h, _):
        base = h * 2 * K
        gdsA = [pltpu.async_copy(table_hbm.at[src_v.at[base + b]],
                                 rows_v.at[0, b], gsemA) for b in range(K)]
        gdsB = [pltpu.async_copy(table_hbm.at[src_v.at[base + K + b]],
                                 rows_v.at[1, b], gsemB) for b in range(K)]
        for d in gdsA:
            d.wait()
        sdsA = [pltpu.async_copy(rows_v.at[0, b],
                                 acc_sh.at[dst_v.at[base + b]],
                                 ssem, add=True) for b in range(K)]
        for d in gdsB:
            d.wait()
        sdsB = [pltpu.async_copy(rows_v.at[1, b],
                                 acc_sh.at[dst_v.at[base + K + b]],
                                 ssem, add=True) for b in range(K)]
        for d in sdsA + sdsB:
            d.wait()
        return 0
    lax.fori_loop(0, G // 2, _pair, 0)

    plsc.subcore_barrier()
    pltpu.sync_copy(acc_sh.at[pl.ds(s * RPT, RPT)],
                    parts_hbm.at[c, pl.ds(s * RPT, RPT)])


def _proj_body(x_ref, w_ref, o_ref):
    o_ref[...] = jnp.dot(x_ref[...], w_ref[...],
                         preferred_element_type=jnp.float32)


_proj = pl.pallas_call(
    _proj_body,
    grid=(GRID,),
    in_specs=[
        pl.BlockSpec((BLK, F_IN), lambda k: (k, 0)),
        pl.BlockSpec((F_IN, DPAD), lambda k: (0, 0)),
    ],
    out_specs=pl.BlockSpec((BLK, DPAD), lambda k: (k, 0)),
    out_shape=jax.ShapeDtypeStruct((N, DPAD), jnp.float32),
)


def _mlp_body(u_ref, parts_ref, ba_ref, wb_ref, bb_ref, wa2_ref,
              unext_ref, s_ref):
    k = pl.program_id(0)
    pre = u_ref[...] + parts_ref[0] + parts_ref[1] + ba_ref[...]
    h = jnp.maximum(pre, 0.0)
    t = jnp.dot(h, wb_ref[...], preferred_element_type=jnp.float32) + bb_ref[...]
    xv = jnp.maximum(t, 0.0)
    unext_ref[...] = jnp.dot(xv, wa2_ref[...],
                             preferred_element_type=jnp.float32)

    @pl.when(k == 0)
    def _():
        s_ref[...] = jnp.zeros_like(s_ref)

    s_ref[...] += jnp.sum(xv, axis=0, keepdims=True)


_mlp = pl.pallas_call(
    _mlp_body,
    grid=(GRID,),
    in_specs=[
        pl.BlockSpec((BLK, DPAD), lambda k: (k, 0)),
        pl.BlockSpec((2, BLK, DPAD), lambda k: (0, k, 0)),
        pl.BlockSpec((1, DPAD), lambda k: (0, 0)),
        pl.BlockSpec((DPAD, DPAD), lambda k: (0, 0)),
        pl.BlockSpec((1, DPAD), lambda k: (0, 0)),
        pl.BlockSpec((DPAD, DPAD), lambda k: (0, 0)),
    ],
    out_specs=[
        pl.BlockSpec((BLK, DPAD), lambda k: (k, 0)),
        pl.BlockSpec((1, DPAD), lambda k: (0, 0)),
    ],
    out_shape=[
        jax.ShapeDtypeStruct((N, DPAD), jnp.float32),
        jax.ShapeDtypeStruct((1, DPAD), jnp.float32),
    ],
)


def _mlp_last_body(u_ref, parts_ref, ba_ref, wb_ref, bb_ref, s_ref):
    k = pl.program_id(0)
    pre = u_ref[...] + parts_ref[0] + parts_ref[1] + ba_ref[...]
    h = jnp.maximum(pre, 0.0)
    t = jnp.dot(h, wb_ref[...], preferred_element_type=jnp.float32) + bb_ref[...]
    xv = jnp.maximum(t, 0.0)

    @pl.when(k == 0)
    def _():
        s_ref[...] = jnp.zeros_like(s_ref)

    s_ref[...] += jnp.sum(xv, axis=0, keepdims=True)


_mlp_last = pl.pallas_call(
    _mlp_last_body,
    grid=(GRID,),
    in_specs=[
        pl.BlockSpec((BLK, DPAD), lambda k: (k, 0)),
        pl.BlockSpec((2, BLK, DPAD), lambda k: (0, k, 0)),
        pl.BlockSpec((1, DPAD), lambda k: (0, 0)),
        pl.BlockSpec((DPAD, DPAD), lambda k: (0, 0)),
        pl.BlockSpec((1, DPAD), lambda k: (0, 0)),
    ],
    out_specs=pl.BlockSpec((1, DPAD), lambda k: (0, 0)),
    out_shape=jax.ShapeDtypeStruct((1, DPAD), jnp.float32),
)


def _final_body(s_ref, l_ref, o_ref):
    tot = jnp.sum(s_ref[...] * l_ref[...]) * (1.0 / N)
    o_ref[...] = jax.nn.sigmoid(tot).reshape(1, 1)


_final = pl.pallas_call(
    _final_body,
    in_specs=[
        pl.BlockSpec((8, DPAD), lambda: (0, 0)),
        pl.BlockSpec((8, DPAD), lambda: (0, 0)),
    ],
    out_specs=pl.BlockSpec((1, 1), lambda: (0, 0)),
    out_shape=jax.ShapeDtypeStruct((1, 1), jnp.float32),
)


def kernel(x, edge_index,
           W1a, b1a, W1b, b1b,
           W2a, b2a, W2b, b2b,
           W3a, b3a, W3b, b3b,
           W4a, b4a, W4b, b4b,
           W5a, b5a, W5b, b5b,
           L1, L2, L3, L4, L5):
    src3 = edge_index[0].reshape(NW, NCHUNK, C)
    dst3 = edge_index[1].reshape(NW, NCHUNK, C)

    W1a_p = _pad_mat(W1a, F_IN, DPAD)
    Was = [None, _pad_mat(W2a, DPAD, DPAD), _pad_mat(W3a, DPAD, DPAD),
           _pad_mat(W4a, DPAD, DPAD), _pad_mat(W5a, DPAD, DPAD)]
    Wbs = [_pad_mat(W1b, DPAD, DPAD), _pad_mat(W2b, DPAD, DPAD),
           _pad_mat(W3b, DPAD, DPAD), _pad_mat(W4b, DPAD, DPAD),
           _pad_mat(W5b, DPAD, DPAD)]
    bas = [_pad_vec(b1a, DPAD), _pad_vec(b2a, DPAD), _pad_vec(b3a, DPAD),
           _pad_vec(b4a, DPAD), _pad_vec(b5a, DPAD)]
    bbs = [_pad_vec(b1b, DPAD), _pad_vec(b2b, DPAD), _pad_vec(b3b, DPAD),
           _pad_vec(b4b, DPAD), _pad_vec(b5b, DPAD)]

    u = _proj(x, W1a_p)
    sums = []
    for i in range(5):
        parts = _sc_agg(u, src3, dst3)
        if i < 4:
            u, s_i = _mlp(u, parts, bas[i], Wbs[i], bbs[i], Was[i + 1])
        else:
            s_i = _mlp_last(u, parts, bas[i], Wbs[i], bbs[i])
        sums.append(s_i)

    S = jnp.concatenate(sums + [jnp.zeros((3, DPAD), jnp.float32)], axis=0)
    Lrows = jnp.stack([_pad_vec(L[:, 0], DPAD)[0]
                       for L in (L1, L2, L3, L4, L5)], axis=0)
    Lp = jnp.concatenate([Lrows, jnp.zeros((3, DPAD), jnp.float32)], axis=0)
    return _final(S, Lp)


# speedup vs baseline: 24.3754x; 1.0857x over previous
import functools

import jax
import jax.numpy as jnp
from jax import lax
from jax.experimental import pallas as pl
from jax.experimental.pallas import tpu as pltpu
import jax.experimental.pallas.tpu_sc as plsc

N = 10000
E = 320000
F_IN = 128
DIM = 10
DPAD = 16

NW = 32
EPW = E // NW
C = 125
NCHUNK = EPW // C
K = 8
G = NCHUNK // K
NACC = 10240
RPT = NACC // 16
ZC = 128

BLK = 1000
GRID = N // BLK


def _pad_mat(w, rows, cols):
    return jnp.zeros((rows, cols), jnp.float32).at[: w.shape[0], : w.shape[1]].set(w)


def _pad_vec(b, cols):
    return jnp.zeros((1, cols), jnp.float32).at[0, : b.shape[0]].set(b)


_MESH = plsc.VectorSubcoreMesh(core_axis_name="c", subcore_axis_name="s")


@functools.partial(
    pl.kernel,
    out_type=jax.ShapeDtypeStruct((2, NACC, DPAD), jnp.float32),
    mesh=_MESH,
    scratch_types=[
        pltpu.VMEM((NCHUNK, C), jnp.int32),
        pltpu.VMEM((NCHUNK, C), jnp.int32),
        pltpu.VMEM((2, K, C, DPAD), jnp.float32),
        pltpu.VMEM((ZC, DPAD), jnp.float32),
        pltpu.VMEM_SHARED((NACC, DPAD), jnp.float32),
        pltpu.SemaphoreType.DMA,
        pltpu.SemaphoreType.DMA,
        pltpu.SemaphoreType.DMA,
    ],
    compiler_params=pltpu.CompilerParams(use_tc_tiling_on_sc=False),
)
def _sc_agg(table_hbm, src_hbm, dst_hbm, parts_hbm,
            src_v, dst_v, rows_v, zero_v, acc_sh, gsemA, gsemB, ssem):
    c = lax.axis_index("c")
    s = lax.axis_index("s")
    wid = c * 16 + s

    pltpu.sync_copy(src_hbm.at[wid], src_v)
    pltpu.sync_copy(dst_hbm.at[wid], dst_v)

    def _zero_row(i, _):
        zero_v[i, :] = jnp.zeros((DPAD,), jnp.float32)
        return 0
    lax.fori_loop(0, ZC, _zero_row, 0)

    def _zero_stripe(j, _):
        pltpu.sync_copy(zero_v, acc_sh.at[pl.ds(s * RPT + j * ZC, ZC)])
        return 0
    lax.fori_loop(0, RPT // ZC, _zero_stripe, 0)

    plsc.subcore_barrier()

    def _pair(h, _):
        base = h * 2 * K
        gdsA = [pltpu.async_copy(table_hbm.at[src_v.at[base + b]],
                                 rows_v.at[0, b], gsemA) for b in range(K)]
        gdsB = [pltpu.async_copy(table_hbm.at[src_v.at[base + K + b]],
                                 rows_v.at[1, b], gsemB) for b in range(K)]
        for d in gdsA:
            d.wait()
        sdsA = [pltpu.async_copy(rows_v.at[0, b],
                                 acc_sh.at[dst_v.at[base + b]],
                                 ssem, add=True) for b in range(K)]
        for d in gdsB:
            d.wait()
        sdsB = [pltpu.async_copy(rows_v.at[1, b],
                                 acc_sh.at[dst_v.at[base + K + b]],
                                 ssem, add=True) for b in range(K)]
        for d in sdsA + sdsB:
            d.wait()
        return 0
    lax.fori_loop(0, G // 2, _pair, 0)

    plsc.subcore_barrier()
    pltpu.sync_copy(acc_sh.at[pl.ds(s * RPT, RPT)],
                    parts_hbm.at[c, pl.ds(s * RPT, RPT)])


def _proj_body(x_ref, w_ref, o_ref):
    o_ref[...] = jnp.dot(x_ref[...], w_ref[...],
                         preferred_element_type=jnp.float32)


_proj = pl.pallas_call(
    _proj_body,
    in_specs=[
        pl.BlockSpec((N, F_IN), lambda: (0, 0)),
        pl.BlockSpec((F_IN, DPAD), lambda: (0, 0)),
    ],
    out_specs=pl.BlockSpec((N, DPAD), lambda: (0, 0)),
    out_shape=jax.ShapeDtypeStruct((N, DPAD), jnp.float32),
)


def _mlp_body(u_ref, parts_ref, ba_ref, wb_ref, bb_ref, wa2_ref,
              unext_ref, s_ref):
    pre = u_ref[...] + parts_ref[0, :N] + parts_ref[1, :N] + ba_ref[...]
    h = jnp.maximum(pre, 0.0)
    t = jnp.dot(h, wb_ref[...], preferred_element_type=jnp.float32) + bb_ref[...]
    xv = jnp.maximum(t, 0.0)
    unext_ref[...] = jnp.dot(xv, wa2_ref[...],
                             preferred_element_type=jnp.float32)
    s_ref[...] = jnp.sum(xv, axis=0, keepdims=True)


_mlp = pl.pallas_call(
    _mlp_body,
    in_specs=[
        pl.BlockSpec((N, DPAD), lambda: (0, 0)),
        pl.BlockSpec((2, NACC, DPAD), lambda: (0, 0, 0)),
        pl.BlockSpec((1, DPAD), lambda: (0, 0)),
        pl.BlockSpec((DPAD, DPAD), lambda: (0, 0)),
        pl.BlockSpec((1, DPAD), lambda: (0, 0)),
        pl.BlockSpec((DPAD, DPAD), lambda: (0, 0)),
    ],
    out_specs=[
        pl.BlockSpec((N, DPAD), lambda: (0, 0)),
        pl.BlockSpec((1, DPAD), lambda: (0, 0)),
    ],
    out_shape=[
        jax.ShapeDtypeStruct((N, DPAD), jnp.float32),
        jax.ShapeDtypeStruct((1, DPAD), jnp.float32),
    ],
)


def _mlp_last_body(u_ref, parts_ref, ba_ref, wb_ref, bb_ref,
                   sprev_ref, l_ref, o_ref):
    pre = u_ref[...] + parts_ref[0, :N] + parts_ref[1, :N] + ba_ref[...]
    h = jnp.maximum(pre, 0.0)
    t = jnp.dot(h, wb_ref[...], preferred_element_type=jnp.float32) + bb_ref[...]
    xv = jnp.maximum(t, 0.0)
    s5 = jnp.sum(xv, axis=0, keepdims=True)
    tot = (jnp.sum(sprev_ref[...] * l_ref[0:4]) +
           jnp.sum(s5 * l_ref[4:5])) * (1.0 / N)
    o_ref[...] = jax.nn.sigmoid(tot).reshape(1, 1)


_mlp_last = pl.pallas_call(
    _mlp_last_body,
    in_specs=[
        pl.BlockSpec((N, DPAD), lambda: (0, 0)),
        pl.BlockSpec((2, NACC, DPAD), lambda: (0, 0, 0)),
        pl.BlockSpec((1, DPAD), lambda: (0, 0)),
        pl.BlockSpec((DPAD, DPAD), lambda: (0, 0)),
        pl.BlockSpec((1, DPAD), lambda: (0, 0)),
        pl.BlockSpec((4, DPAD), lambda: (0, 0)),
        pl.BlockSpec((8, DPAD), lambda: (0, 0)),
    ],
    out_specs=pl.BlockSpec((1, 1), lambda: (0, 0)),
    out_shape=jax.ShapeDtypeStruct((1, 1), jnp.float32),
)


def kernel(x, edge_index,
           W1a, b1a, W1b, b1b,
           W2a, b2a, W2b, b2b,
           W3a, b3a, W3b, b3b,
           W4a, b4a, W4b, b4b,
           W5a, b5a, W5b, b5b,
           L1, L2, L3, L4, L5):
    src3 = edge_index[0].reshape(NW, NCHUNK, C)
    dst3 = edge_index[1].reshape(NW, NCHUNK, C)

    W1a_p = _pad_mat(W1a, F_IN, DPAD)
    Was = [None, _pad_mat(W2a, DPAD, DPAD), _pad_mat(W3a, DPAD, DPAD),
           _pad_mat(W4a, DPAD, DPAD), _pad_mat(W5a, DPAD, DPAD)]
    Wbs = [_pad_mat(W1b, DPAD, DPAD), _pad_mat(W2b, DPAD, DPAD),
           _pad_mat(W3b, DPAD, DPAD), _pad_mat(W4b, DPAD, DPAD),
           _pad_mat(W5b, DPAD, DPAD)]
    bas = [_pad_vec(b1a, DPAD), _pad_vec(b2a, DPAD), _pad_vec(b3a, DPAD),
           _pad_vec(b4a, DPAD), _pad_vec(b5a, DPAD)]
    bbs = [_pad_vec(b1b, DPAD), _pad_vec(b2b, DPAD), _pad_vec(b3b, DPAD),
           _pad_vec(b4b, DPAD), _pad_vec(b5b, DPAD)]

    Lrows = jnp.stack([_pad_vec(L[:, 0], DPAD)[0]
                       for L in (L1, L2, L3, L4, L5)], axis=0)
    Lp = jnp.concatenate([Lrows, jnp.zeros((3, DPAD), jnp.float32)], axis=0)

    u = _proj(x, W1a_p)
    sums = []
    for i in range(4):
        parts = _sc_agg(u, src3, dst3)
        u, s_i = _mlp(u, parts, bas[i], Wbs[i], bbs[i], Was[i + 1])
        sums.append(s_i)
    parts = _sc_agg(u, src3, dst3)
    sprev = jnp.concatenate(sums, axis=0)
    return _mlp_last(u, parts, bas[4], Wbs[4], bbs[4], sprev, Lp)


# speedup vs baseline: 37.2608x; 1.5286x over previous
import functools

import jax
import jax.numpy as jnp
from jax import lax
from jax.experimental import pallas as pl
from jax.experimental.pallas import tpu as pltpu
import jax.experimental.pallas.tpu_sc as plsc

N = 10000
E = 320000
F_IN = 128
DIM = 10
DPAD = 16

NW = 32
EPW = E // NW
C = 125
NCHUNK = EPW // C
K = 8
G = NCHUNK // K
NACC = 10240
RPT = NACC // 16
ZC = 128

PK = 8
NP = NACC // PK
NRP = N // PK
W128 = PK * DPAD


def _pad_mat(w, rows, cols):
    return jnp.zeros((rows, cols), jnp.float32).at[: w.shape[0], : w.shape[1]].set(w)


def _pad_vec(b, cols):
    return jnp.zeros((1, cols), jnp.float32).at[0, : b.shape[0]].set(b)


def _blockdiag(w):
    return jnp.kron(jnp.eye(PK, dtype=jnp.float32), _pad_mat(w, DPAD, DPAD))


def _tile_vec(b):
    return jnp.tile(_pad_vec(b, DPAD), (1, PK))


_MESH = plsc.VectorSubcoreMesh(core_axis_name="c", subcore_axis_name="s")


@functools.partial(
    pl.kernel,
    out_type=jax.ShapeDtypeStruct((2, NACC, DPAD), jnp.float32),
    mesh=_MESH,
    scratch_types=[
        pltpu.VMEM((NCHUNK, C), jnp.int32),
        pltpu.VMEM((NCHUNK, C), jnp.int32),
        pltpu.VMEM((2, K, C, DPAD), jnp.float32),
        pltpu.VMEM((ZC, DPAD), jnp.float32),
        pltpu.VMEM_SHARED((NACC, DPAD), jnp.float32),
        pltpu.SemaphoreType.DMA,
        pltpu.SemaphoreType.DMA,
        pltpu.SemaphoreType.DMA,
    ],
    compiler_params=pltpu.CompilerParams(use_tc_tiling_on_sc=False),
)
def _sc_agg(table_hbm, ei_hbm, parts_hbm,
            src_v, dst_v, rows_v, zero_v, acc_sh, gsemA, gsemB, ssem):
    c = lax.axis_index("c")
    s = lax.axis_index("s")
    wid = c * 16 + s

    pltpu.sync_copy(ei_hbm.at[0, wid], src_v)
    pltpu.sync_copy(ei_hbm.at[1, wid], dst_v)

    def _zero_row(i, _):
        zero_v[i, :] = jnp.zeros((DPAD,), jnp.float32)
        return 0
    lax.fori_loop(0, ZC, _zero_row, 0)

    def _zero_stripe(j, _):
        pltpu.sync_copy(zero_v, acc_sh.at[pl.ds(s * RPT + j * ZC, ZC)])
        return 0
    lax.fori_loop(0, RPT // ZC, _zero_stripe, 0)

    plsc.subcore_barrier()

    def _pair(h, _):
        base = h * 2 * K
        gdsA = [pltpu.async_copy(table_hbm.at[src_v.at[base + b]],
                                 rows_v.at[0, b], gsemA) for b in range(K)]
        gdsB = [pltpu.async_copy(table_hbm.at[src_v.at[base + K + b]],
                                 rows_v.at[1, b], gsemB) for b in range(K)]
        for d in gdsA:
            d.wait()
        sdsA = [pltpu.async_copy(rows_v.at[0, b],
                                 acc_sh.at[dst_v.at[base + b]],
                                 ssem, add=True) for b in range(K)]
        for d in gdsB:
            d.wait()
        sdsB = [pltpu.async_copy(rows_v.at[1, b],
                                 acc_sh.at[dst_v.at[base + K + b]],
                                 ssem, add=True) for b in range(K)]
        for d in sdsA + sdsB:
            d.wait()
        return 0
    lax.fori_loop(0, G // 2, _pair, 0)

    plsc.subcore_barrier()
    pltpu.sync_copy(acc_sh.at[pl.ds(s * RPT, RPT)],
                    parts_hbm.at[c, pl.ds(s * RPT, RPT)])


def _proj_body(x_ref, w_ref, o_ref):
    u = jnp.dot(x_ref[...], w_ref[...], preferred_element_type=jnp.float32)
    o_ref[...] = jnp.concatenate(
        [u, jnp.zeros((NP - NRP, W128), jnp.float32)], axis=0)


_proj = pl.pallas_call(
    _proj_body,
    in_specs=[
        pl.BlockSpec((NRP, PK * F_IN), lambda: (0, 0)),
        pl.BlockSpec((PK * F_IN, W128), lambda: (0, 0)),
    ],
    out_specs=pl.BlockSpec((NP, W128), lambda: (0, 0)),
    out_shape=jax.ShapeDtypeStruct((NP, W128), jnp.float32),
)


def _row_mask(v):
    rows = lax.broadcasted_iota(jnp.int32, (NP, W128), 0)
    return jnp.where(rows < NRP, v, 0.0)


def _mlp_body(u_ref, parts_ref, ba_ref, wb_ref, bb_ref, wa2_ref,
              unext_ref, s_ref):
    pre = u_ref[...] + parts_ref[0] + parts_ref[1] + ba_ref[...]
    h = jnp.maximum(pre, 0.0)
    t = jnp.dot(h, wb_ref[...], preferred_element_type=jnp.float32) + bb_ref[...]
    xv = _row_mask(jnp.maximum(t, 0.0))
    unext_ref[...] = jnp.dot(xv, wa2_ref[...],
                             preferred_element_type=jnp.float32)
    s_ref[...] = jnp.sum(xv, axis=0, keepdims=True)


_mlp = pl.pallas_call(
    _mlp_body,
    in_specs=[
        pl.BlockSpec((NP, W128), lambda: (0, 0)),
        pl.BlockSpec((2, NP, W128), lambda: (0, 0, 0)),
        pl.BlockSpec((1, W128), lambda: (0, 0)),
        pl.BlockSpec((W128, W128), lambda: (0, 0)),
        pl.BlockSpec((1, W128), lambda: (0, 0)),
        pl.BlockSpec((W128, W128), lambda: (0, 0)),
    ],
    out_specs=[
        pl.BlockSpec((NP, W128), lambda: (0, 0)),
        pl.BlockSpec((1, W128), lambda: (0, 0)),
    ],
    out_shape=[
        jax.ShapeDtypeStruct((NP, W128), jnp.float32),
        jax.ShapeDtypeStruct((1, W128), jnp.float32),
    ],
)


def _mlp_last_body(u_ref, parts_ref, ba_ref, wb_ref, bb_ref,
                   sprev_ref, l_ref, o_ref):
    pre = u_ref[...] + parts_ref[0] + parts_ref[1] + ba_ref[...]
    h = jnp.maximum(pre, 0.0)
    t = jnp.dot(h, wb_ref[...], preferred_element_type=jnp.float32) + bb_ref[...]
    xv = _row_mask(jnp.maximum(t, 0.0))
    s5 = jnp.sum(xv, axis=0, keepdims=True)
    tot = (jnp.sum(sprev_ref[...] * l_ref[0:4]) +
           jnp.sum(s5 * l_ref[4:5])) * (1.0 / N)
    o_ref[...] = jax.nn.sigmoid(tot).reshape(1, 1)


_mlp_last = pl.pallas_call(
    _mlp_last_body,
    in_specs=[
        pl.BlockSpec((NP, W128), lambda: (0, 0)),
        pl.BlockSpec((2, NP, W128), lambda: (0, 0, 0)),
        pl.BlockSpec((1, W128), lambda: (0, 0)),
        pl.BlockSpec((W128, W128), lambda: (0, 0)),
        pl.BlockSpec((1, W128), lambda: (0, 0)),
        pl.BlockSpec((4, W128), lambda: (0, 0)),
        pl.BlockSpec((8, W128), lambda: (0, 0)),
    ],
    out_specs=pl.BlockSpec((1, 1), lambda: (0, 0)),
    out_shape=jax.ShapeDtypeStruct((1, 1), jnp.float32),
)


def kernel(x, edge_index,
           W1a, b1a, W1b, b1b,
           W2a, b2a, W2b, b2b,
           W3a, b3a, W3b, b3b,
           W4a, b4a, W4b, b4b,
           W5a, b5a, W5b, b5b,
           L1, L2, L3, L4, L5):
    ei4 = edge_index.reshape(2, NW, NCHUNK, C)

    W1a8 = jnp.kron(jnp.eye(PK, dtype=jnp.float32), _pad_mat(W1a, F_IN, DPAD))
    Was = [None, _blockdiag(W2a), _blockdiag(W3a), _blockdiag(W4a),
           _blockdiag(W5a)]
    Wbs = [_blockdiag(W1b), _blockdiag(W2b), _blockdiag(W3b),
           _blockdiag(W4b), _blockdiag(W5b)]
    bas = [_tile_vec(b1a), _tile_vec(b2a), _tile_vec(b3a), _tile_vec(b4a),
           _tile_vec(b5a)]
    bbs = [_tile_vec(b1b), _tile_vec(b2b), _tile_vec(b3b), _tile_vec(b4b),
           _tile_vec(b5b)]

    Lrows = jnp.stack([jnp.tile(_pad_vec(L[:, 0], DPAD), (1, PK))[0]
                       for L in (L1, L2, L3, L4, L5)], axis=0)
    Lp = jnp.concatenate([Lrows, jnp.zeros((3, W128), jnp.float32)], axis=0)

    u = _proj(x.reshape(NRP, PK * F_IN), W1a8)
    sums = []
    for i in range(4):
        parts = _sc_agg(u.reshape(NACC, DPAD), ei4)
        u, s_i = _mlp(u, parts.reshape(2, NP, W128),
                      bas[i], Wbs[i], bbs[i], Was[i + 1])
        sums.append(s_i)
    parts = _sc_agg(u.reshape(NACC, DPAD), ei4)
    sprev = jnp.concatenate(sums, axis=0)
    return _mlp_last(u, parts.reshape(2, NP, W128),
                     bas[4], Wbs[4], bbs[4], sprev, Lp)


# speedup vs baseline: 42.0283x; 1.1280x over previous
import functools

import jax
import jax.numpy as jnp
from jax import lax
from jax.experimental import pallas as pl
from jax.experimental.pallas import tpu as pltpu
import jax.experimental.pallas.tpu_sc as plsc

N = 10000
E = 320000
F_IN = 128
DIM = 10
DPAD = 16

NW = 32
EPW = E // NW
C = 125
NCHUNK = EPW // C
K = 8
G = NCHUNK // K
NACC = 10240
RPT = NACC // 16
ZC = 128

PK = 8
NP = NACC // PK
NRP = N // PK
W128 = PK * DPAD


def _pad_mat(w, rows, cols):
    return jnp.zeros((rows, cols), jnp.float32).at[: w.shape[0], : w.shape[1]].set(w)


def _pad_vec(b, cols):
    return jnp.zeros((1, cols), jnp.float32).at[0, : b.shape[0]].set(b)


def _blockdiag(w):
    return jnp.kron(jnp.eye(PK, dtype=jnp.float32), _pad_mat(w, DPAD, DPAD))


def _tile_vec(b):
    return jnp.tile(_pad_vec(b, DPAD), (1, PK))


_MESH = plsc.VectorSubcoreMesh(core_axis_name="c", subcore_axis_name="s")


@functools.partial(
    pl.kernel,
    out_type=jax.ShapeDtypeStruct((2, NACC, DPAD), jnp.float32),
    mesh=_MESH,
    scratch_types=[
        pltpu.VMEM((NCHUNK, C), jnp.int32),
        pltpu.VMEM((NCHUNK, C), jnp.int32),
        pltpu.VMEM((2, K, C, DPAD), jnp.float32),
        pltpu.VMEM((ZC, DPAD), jnp.float32),
        pltpu.VMEM_SHARED((NACC, DPAD), jnp.float32),
        pltpu.VMEM_SHARED((NACC, DPAD), jnp.float32),
        pltpu.SemaphoreType.DMA,
        pltpu.SemaphoreType.DMA,
        pltpu.SemaphoreType.DMA,
    ],
    compiler_params=pltpu.CompilerParams(use_tc_tiling_on_sc=False),
)
def _sc_agg(table_hbm, ei_hbm, parts_hbm,
            src_v, dst_v, rows_v, zero_v, acc_sh, tab_sh,
            gsemA, gsemB, ssem):
    c = lax.axis_index("c")
    s = lax.axis_index("s")
    wid = c * 16 + s

    pltpu.sync_copy(table_hbm.at[pl.ds(s * RPT, RPT)],
                    tab_sh.at[pl.ds(s * RPT, RPT)])
    pltpu.sync_copy(ei_hbm.at[0, wid], src_v)
    pltpu.sync_copy(ei_hbm.at[1, wid], dst_v)

    def _zero_row(i, _):
        zero_v[i, :] = jnp.zeros((DPAD,), jnp.float32)
        return 0
    lax.fori_loop(0, ZC, _zero_row, 0)

    def _zero_stripe(j, _):
        pltpu.sync_copy(zero_v, acc_sh.at[pl.ds(s * RPT + j * ZC, ZC)])
        return 0
    lax.fori_loop(0, RPT // ZC, _zero_stripe, 0)

    plsc.subcore_barrier()

    def _pair(h, _):
        base = h * 2 * K
        gdsA = [pltpu.async_copy(tab_sh.at[src_v.at[base + b]],
                                 rows_v.at[0, b], gsemA) for b in range(K)]
        gdsB = [pltpu.async_copy(tab_sh.at[src_v.at[base + K + b]],
                                 rows_v.at[1, b], gsemB) for b in range(K)]
        for d in gdsA:
            d.wait()
        sdsA = [pltpu.async_copy(rows_v.at[0, b],
                                 acc_sh.at[dst_v.at[base + b]],
                                 ssem, add=True) for b in range(K)]
        for d in gdsB:
            d.wait()
        sdsB = [pltpu.async_copy(rows_v.at[1, b],
                                 acc_sh.at[dst_v.at[base + K + b]],
                                 ssem, add=True) for b in range(K)]
        for d in sdsA + sdsB:
            d.wait()
        return 0
    lax.fori_loop(0, G // 2, _pair, 0)

    plsc.subcore_barrier()
    pltpu.sync_copy(acc_sh.at[pl.ds(s * RPT, RPT)],
                    parts_hbm.at[c, pl.ds(s * RPT, RPT)])


def _proj_body(x_ref, w_ref, o_ref):
    u = jnp.dot(x_ref[...], w_ref[...], preferred_element_type=jnp.float32)
    o_ref[...] = jnp.concatenate(
        [u, jnp.zeros((NP - NRP, W128), jnp.float32)], axis=0)


_proj = pl.pallas_call(
    _proj_body,
    in_specs=[
        pl.BlockSpec((NRP, PK * F_IN), lambda: (0, 0)),
        pl.BlockSpec((PK * F_IN, W128), lambda: (0, 0)),
    ],
    out_specs=pl.BlockSpec((NP, W128), lambda: (0, 0)),
    out_shape=jax.ShapeDtypeStruct((NP, W128), jnp.float32),
)


def _row_mask(v):
    rows = lax.broadcasted_iota(jnp.int32, (NP, W128), 0)
    return jnp.where(rows < NRP, v, 0.0)


def _mlp_body(u_ref, parts_ref, ba_ref, wb_ref, bb_ref, wa2_ref,
              unext_ref, s_ref):
    pre = u_ref[...] + parts_ref[0] + parts_ref[1] + ba_ref[...]
    h = jnp.maximum(pre, 0.0)
    t = jnp.dot(h, wb_ref[...], preferred_element_type=jnp.float32) + bb_ref[...]
    xv = _row_mask(jnp.maximum(t, 0.0))
    unext_ref[...] = jnp.dot(xv, wa2_ref[...],
                             preferred_element_type=jnp.float32)
    s_ref[...] = jnp.sum(xv, axis=0, keepdims=True)


_mlp = pl.pallas_call(
    _mlp_body,
    in_specs=[
        pl.BlockSpec((NP, W128), lambda: (0, 0)),
        pl.BlockSpec((2, NP, W128), lambda: (0, 0, 0)),
        pl.BlockSpec((1, W128), lambda: (0, 0)),
        pl.BlockSpec((W128, W128), lambda: (0, 0)),
        pl.BlockSpec((1, W128), lambda: (0, 0)),
        pl.BlockSpec((W128, W128), lambda: (0, 0)),
    ],
    out_specs=[
        pl.BlockSpec((NP, W128), lambda: (0, 0)),
        pl.BlockSpec((1, W128), lambda: (0, 0)),
    ],
    out_shape=[
        jax.ShapeDtypeStruct((NP, W128), jnp.float32),
        jax.ShapeDtypeStruct((1, W128), jnp.float32),
    ],
)


def _mlp_last_body(u_ref, parts_ref, ba_ref, wb_ref, bb_ref,
                   sprev_ref, l_ref, o_ref):
    pre = u_ref[...] + parts_ref[0] + parts_ref[1] + ba_ref[...]
    h = jnp.maximum(pre, 0.0)
    t = jnp.dot(h, wb_ref[...], preferred_element_type=jnp.float32) + bb_ref[...]
    xv = _row_mask(jnp.maximum(t, 0.0))
    s5 = jnp.sum(xv, axis=0, keepdims=True)
    tot = (jnp.sum(sprev_ref[...] * l_ref[0:4]) +
           jnp.sum(s5 * l_ref[4:5])) * (1.0 / N)
    o_ref[...] = jax.nn.sigmoid(tot).reshape(1, 1)


_mlp_last = pl.pallas_call(
    _mlp_last_body,
    in_specs=[
        pl.BlockSpec((NP, W128), lambda: (0, 0)),
        pl.BlockSpec((2, NP, W128), lambda: (0, 0, 0)),
        pl.BlockSpec((1, W128), lambda: (0, 0)),
        pl.BlockSpec((W128, W128), lambda: (0, 0)),
        pl.BlockSpec((1, W128), lambda: (0, 0)),
        pl.BlockSpec((4, W128), lambda: (0, 0)),
        pl.BlockSpec((8, W128), lambda: (0, 0)),
    ],
    out_specs=pl.BlockSpec((1, 1), lambda: (0, 0)),
    out_shape=jax.ShapeDtypeStruct((1, 1), jnp.float32),
)


def kernel(x, edge_index,
           W1a, b1a, W1b, b1b,
           W2a, b2a, W2b, b2b,
           W3a, b3a, W3b, b3b,
           W4a, b4a, W4b, b4b,
           W5a, b5a, W5b, b5b,
           L1, L2, L3, L4, L5):
    ei4 = edge_index.reshape(2, NW, NCHUNK, C)

    W1a8 = jnp.kron(jnp.eye(PK, dtype=jnp.float32), _pad_mat(W1a, F_IN, DPAD))
    Was = [None, _blockdiag(W2a), _blockdiag(W3a), _blockdiag(W4a),
           _blockdiag(W5a)]
    Wbs = [_blockdiag(W1b), _blockdiag(W2b), _blockdiag(W3b),
           _blockdiag(W4b), _blockdiag(W5b)]
    bas = [_tile_vec(b1a), _tile_vec(b2a), _tile_vec(b3a), _tile_vec(b4a),
           _tile_vec(b5a)]
    bbs = [_tile_vec(b1b), _tile_vec(b2b), _tile_vec(b3b), _tile_vec(b4b),
           _tile_vec(b5b)]

    Lrows = jnp.stack([jnp.tile(_pad_vec(L[:, 0], DPAD), (1, PK))[0]
                       for L in (L1, L2, L3, L4, L5)], axis=0)
    Lp = jnp.concatenate([Lrows, jnp.zeros((3, W128), jnp.float32)], axis=0)

    u = _proj(x.reshape(NRP, PK * F_IN), W1a8)
    sums = []
    for i in range(4):
        parts = _sc_agg(u.reshape(NACC, DPAD), ei4)
        u, s_i = _mlp(u, parts.reshape(2, NP, W128),
                      bas[i], Wbs[i], bbs[i], Was[i + 1])
        sums.append(s_i)
    parts = _sc_agg(u.reshape(NACC, DPAD), ei4)
    sprev = jnp.concatenate(sums, axis=0)
    return _mlp_last(u, parts.reshape(2, NP, W128),
                     bas[4], Wbs[4], bbs[4], sprev, Lp)


# speedup vs baseline: 44.8017x; 1.0660x over previous
import functools

import jax
import jax.numpy as jnp
from jax import lax
from jax.experimental import pallas as pl
from jax.experimental.pallas import tpu as pltpu
import jax.experimental.pallas.tpu_sc as plsc

N = 10000
E = 320000
F_IN = 128
DIM = 10
DPAD = 16

NW = 32
EPW = E // NW
C = 125
NCHUNK = EPW // C
K = 8
G = NCHUNK // K
NACC = 10240
RPT = NACC // 16
ZC = 128

PK = 8
NP = NACC // PK
NRP = N // PK
W128 = PK * DPAD


def _pad_mat(w, rows, cols):
    return jnp.zeros((rows, cols), jnp.float32).at[: w.shape[0], : w.shape[1]].set(w)


def _pad_vec(b, cols):
    return jnp.zeros((1, cols), jnp.float32).at[0, : b.shape[0]].set(b)


def _blockdiag(w):
    return jnp.kron(jnp.eye(PK, dtype=jnp.float32), _pad_mat(w, DPAD, DPAD))


def _tile_vec(b):
    return jnp.tile(_pad_vec(b, DPAD), (1, PK))


_MESH = plsc.VectorSubcoreMesh(core_axis_name="c", subcore_axis_name="s")


@functools.partial(
    pl.kernel,
    out_type=jax.ShapeDtypeStruct((2, NACC, DPAD), jnp.float32),
    mesh=_MESH,
    scratch_types=[
        pltpu.VMEM((NCHUNK, C), jnp.int32),
        pltpu.VMEM((NCHUNK, C), jnp.int32),
        pltpu.VMEM((2, K, C, DPAD), jnp.float32),
        pltpu.VMEM((ZC, DPAD), jnp.float32),
        pltpu.VMEM_SHARED((NACC, DPAD), jnp.float32),
        pltpu.VMEM_SHARED((NACC, DPAD), jnp.float32),
        pltpu.SemaphoreType.DMA,
        pltpu.SemaphoreType.DMA,
        pltpu.SemaphoreType.DMA,
    ],
    compiler_params=pltpu.CompilerParams(use_tc_tiling_on_sc=False),
)
def _sc_agg(table_hbm, ei_hbm, parts_hbm,
            src_v, dst_v, rows_v, zero_v, acc_sh, tab_sh,
            gsemA, gsemB, ssem):
    c = lax.axis_index("c")
    s = lax.axis_index("s")
    wid = c * 16 + s

    d1 = pltpu.async_copy(table_hbm.at[pl.ds(s * RPT, RPT)],
                          tab_sh.at[pl.ds(s * RPT, RPT)], gsemA)
    d2 = pltpu.async_copy(ei_hbm.at[0, wid], src_v, gsemA)
    d3 = pltpu.async_copy(ei_hbm.at[1, wid], dst_v, gsemA)

    def _zero_row(i, _):
        zero_v[i, :] = jnp.zeros((DPAD,), jnp.float32)
        return 0
    lax.fori_loop(0, ZC, _zero_row, 0)
    d1.wait()
    d2.wait()
    d3.wait()

    def _zero_stripe(j, _):
        pltpu.sync_copy(zero_v, acc_sh.at[pl.ds(s * RPT + j * ZC, ZC)])
        return 0
    lax.fori_loop(0, RPT // ZC, _zero_stripe, 0)

    plsc.subcore_barrier()

    def _pair(h, _):
        base = h * 2 * K
        gdsA = [pltpu.async_copy(tab_sh.at[src_v.at[base + b]],
                                 rows_v.at[0, b], gsemA) for b in range(K)]
        gdsB = [pltpu.async_copy(tab_sh.at[src_v.at[base + K + b]],
                                 rows_v.at[1, b], gsemB) for b in range(K)]
        for d in gdsA:
            d.wait()
        sdsA = [pltpu.async_copy(rows_v.at[0, b],
                                 acc_sh.at[dst_v.at[base + b]],
                                 ssem, add=True) for b in range(K)]
        for d in gdsB:
            d.wait()
        sdsB = [pltpu.async_copy(rows_v.at[1, b],
                                 acc_sh.at[dst_v.at[base + K + b]],
                                 ssem, add=True) for b in range(K)]
        for d in sdsA + sdsB:
            d.wait()
        return 0
    lax.fori_loop(0, G // 2, _pair, 0)

    plsc.subcore_barrier()
    pltpu.sync_copy(acc_sh.at[pl.ds(s * RPT, RPT)],
                    parts_hbm.at[c, pl.ds(s * RPT, RPT)])


def _proj_body(x_ref, w_ref, o_ref):
    u = jnp.dot(x_ref[...], w_ref[...], preferred_element_type=jnp.float32)
    o_ref[...] = jnp.concatenate(
        [u, jnp.zeros((NP - NRP, W128), jnp.float32)], axis=0)


_proj = pl.pallas_call(
    _proj_body,
    in_specs=[
        pl.BlockSpec((NRP, PK * F_IN), lambda: (0, 0)),
        pl.BlockSpec((PK * F_IN, W128), lambda: (0, 0)),
    ],
    out_specs=pl.BlockSpec((NP, W128), lambda: (0, 0)),
    out_shape=jax.ShapeDtypeStruct((NP, W128), jnp.float32),
)


def _row_mask(v):
    rows = lax.broadcasted_iota(jnp.int32, (NP, W128), 0)
    return jnp.where(rows < NRP, v, 0.0)


def _mlp_body(u_ref, parts_ref, ba_ref, wb_ref, bb_ref, wa2_ref,
              unext_ref, s_ref):
    pre = u_ref[...] + parts_ref[0] + parts_ref[1] + ba_ref[...]
    h = jnp.maximum(pre, 0.0)
    t = jnp.dot(h, wb_ref[...], preferred_element_type=jnp.float32) + bb_ref[...]
    xv = _row_mask(jnp.maximum(t, 0.0))
    unext_ref[...] = jnp.dot(xv, wa2_ref[...],
                             preferred_element_type=jnp.float32)
    s_ref[...] = jnp.sum(xv, axis=0, keepdims=True)


_mlp = pl.pallas_call(
    _mlp_body,
    in_specs=[
        pl.BlockSpec((NP, W128), lambda: (0, 0)),
        pl.BlockSpec((2, NP, W128), lambda: (0, 0, 0)),
        pl.BlockSpec((1, W128), lambda: (0, 0)),
        pl.BlockSpec((W128, W128), lambda: (0, 0)),
        pl.BlockSpec((1, W128), lambda: (0, 0)),
        pl.BlockSpec((W128, W128), lambda: (0, 0)),
    ],
    out_specs=[
        pl.BlockSpec((NP, W128), lambda: (0, 0)),
        pl.BlockSpec((1, W128), lambda: (0, 0)),
    ],
    out_shape=[
        jax.ShapeDtypeStruct((NP, W128), jnp.float32),
        jax.ShapeDtypeStruct((1, W128), jnp.float32),
    ],
)


def _mlp_last_body(u_ref, parts_ref, ba_ref, wb_ref, bb_ref,
                   sprev_ref, l_ref, o_ref):
    pre = u_ref[...] + parts_ref[0] + parts_ref[1] + ba_ref[...]
    h = jnp.maximum(pre, 0.0)
    t = jnp.dot(h, wb_ref[...], preferred_element_type=jnp.float32) + bb_ref[...]
    xv = _row_mask(jnp.maximum(t, 0.0))
    s5 = jnp.sum(xv, axis=0, keepdims=True)
    tot = (jnp.sum(sprev_ref[...] * l_ref[0:4]) +
           jnp.sum(s5 * l_ref[4:5])) * (1.0 / N)
    o_ref[...] = jax.nn.sigmoid(tot).reshape(1, 1)


_mlp_last = pl.pallas_call(
    _mlp_last_body,
    in_specs=[
        pl.BlockSpec((NP, W128), lambda: (0, 0)),
        pl.BlockSpec((2, NP, W128), lambda: (0, 0, 0)),
        pl.BlockSpec((1, W128), lambda: (0, 0)),
        pl.BlockSpec((W128, W128), lambda: (0, 0)),
        pl.BlockSpec((1, W128), lambda: (0, 0)),
        pl.BlockSpec((4, W128), lambda: (0, 0)),
        pl.BlockSpec((8, W128), lambda: (0, 0)),
    ],
    out_specs=pl.BlockSpec((1, 1), lambda: (0, 0)),
    out_shape=jax.ShapeDtypeStruct((1, 1), jnp.float32),
)


def kernel(x, edge_index,
           W1a, b1a, W1b, b1b,
           W2a, b2a, W2b, b2b,
           W3a, b3a, W3b, b3b,
           W4a, b4a, W4b, b4b,
           W5a, b5a, W5b, b5b,
           L1, L2, L3, L4, L5):
    ei4 = edge_index.reshape(2, NW, NCHUNK, C)

    W1a8 = jnp.kron(jnp.eye(PK, dtype=jnp.float32), _pad_mat(W1a, F_IN, DPAD))
    Was = [None, _blockdiag(W2a), _blockdiag(W3a), _blockdiag(W4a),
           _blockdiag(W5a)]
    Wbs = [_blockdiag(W1b), _blockdiag(W2b), _blockdiag(W3b),
           _blockdiag(W4b), _blockdiag(W5b)]
    bas = [_tile_vec(b1a), _tile_vec(b2a), _tile_vec(b3a), _tile_vec(b4a),
           _tile_vec(b5a)]
    bbs = [_tile_vec(b1b), _tile_vec(b2b), _tile_vec(b3b), _tile_vec(b4b),
           _tile_vec(b5b)]

    Lrows = jnp.stack([jnp.tile(_pad_vec(L[:, 0], DPAD), (1, PK))[0]
                       for L in (L1, L2, L3, L4, L5)], axis=0)
    Lp = jnp.concatenate([Lrows, jnp.zeros((3, W128), jnp.float32)], axis=0)

    u = _proj(x.reshape(NRP, PK * F_IN), W1a8)
    sums = []
    for i in range(4):
        parts = _sc_agg(u.reshape(NACC, DPAD), ei4)
        u, s_i = _mlp(u, parts.reshape(2, NP, W128),
                      bas[i], Wbs[i], bbs[i], Was[i + 1])
        sums.append(s_i)
    parts = _sc_agg(u.reshape(NACC, DPAD), ei4)
    sprev = jnp.concatenate(sums, axis=0)
    return _mlp_last(u, parts.reshape(2, NP, W128),
                     bas[4], Wbs[4], bbs[4], sprev, Lp)
